# baseline (speedup 1.0000x reference)
import jax
import jax.numpy as jnp
from jax import lax
from jax.experimental import pallas as pl
from jax.experimental.pallas import tpu as pltpu
from jax.experimental.pallas import tpu_sc as plsc

VOCAB = 100000
B = 16384
L = 50
D = 64
LANES = 16
LPAD = 64
LGATH = 56

NC = 2
NS = 16
NW = NC * NS
SPW = B // NW
NBUF = 8


def _flags_body(v_ref, f_ref):
  s = jnp.sum(v_ref[...], axis=0)
  f_ref[...] = jnp.where(s != 0.0, 1.0, 0.0).astype(jnp.float32)


def _tree_push(stack, v):
  rank = 0
  while stack and stack[-1][0] == rank:
    _, u = stack.pop()
    v = u + v
    rank += 1
  stack.append((rank, v))


def _body(xT_hbm, tab_hbm, flags_hbm, out_hbm,
          idxT_v, sidx_v, rows_v, flg_v, out_v, flags_sh, *sems):
  rsems = sems[:NBUF]
  fsems = sems[NBUF:]
  wid = lax.axis_index("s") * NC + lax.axis_index("c")
  base = wid * SPW

  @pl.when(lax.axis_index("s") == 0)
  def _():
    pltpu.sync_copy(flags_hbm, flags_sh)

  pltpu.sync_copy(xT_hbm.at[:, pl.ds(base, SPW)], idxT_v)
  plsc.subcore_barrier()

  lane = lax.iota(jnp.int32, LANES)
  zero = jnp.zeros((LANES,), jnp.float32)
  one = jnp.ones((LANES,), jnp.float32)
  zero_i = jnp.zeros((LANES,), jnp.int32)

  def build_sidx(s, slot):
    col = zero_i + s
    for k in range(LPAD // LANES):
      row = jnp.minimum(k * LANES + lane, L - 1)
      sidx_v[slot, pl.ds(k * LANES, LANES)] = plsc.load_gather(
          idxT_v, [row, col])

  def fire(s, slot):
    build_sidx(s, slot)
    sidx = sidx_v.at[slot, pl.ds(0, LGATH)]
    pltpu.async_copy(tab_hbm.at[sidx], rows_v.at[slot], rsems[slot])
    pltpu.async_copy(
        flags_sh.at[sidx], flg_v.at[slot, pl.ds(0, LGATH)], fsems[slot])

  def wait(slot):
    sidx = sidx_v.at[slot, pl.ds(0, LGATH)]
    pltpu.make_async_copy(
        tab_hbm.at[sidx], rows_v.at[slot], rsems[slot]).wait()
    pltpu.make_async_copy(
        flags_sh.at[sidx], flg_v.at[slot, pl.ds(0, LGATH)],
        fsems[slot]).wait()

  def compute(s, slot):
    wait(slot)
    flg = flg_v.at[slot]
    g0 = flg[pl.ds(0, LANES)]
    g1 = flg[pl.ds(LANES, LANES)]
    g2 = flg[pl.ds(2 * LANES, LANES)]
    g3 = jnp.where(lane < L - 3 * LANES, flg[pl.ds(3 * LANES, LANES)], zero)
    cs = plsc.cumsum((g0 + g1) + (g2 + g3))
    inv = one / plsc.cummax(lax.rev(cs, (0,)))

    r = rows_v.at[slot]
    stacks = ([], [])
    for l in range(L):
      _tree_push(stacks[0], plsc.bitcast(r[l, pl.ds(0, LANES)], jnp.bfloat16))
      _tree_push(stacks[1],
                 plsc.bitcast(r[l, pl.ds(LANES, 2 * LANES - LANES)],
                              jnp.bfloat16))
    halves = []
    for st in stacks:
      acc = st[0][1]
      for _, v in st[1:]:
        acc = acc + v
      halves.append(acc)

    orow = out_v.at[s]
    for h in range(2):
      ev, od = plsc.unpack(halves[h], format=plsc.PackFormat.INTERLEAVED)
      plsc.store_scatter(orow, [2 * lane + h * 32], ev * inv)
      plsc.store_scatter(orow, [2 * lane + (h * 32 + 1)], od * inv)

  for b_ in range(NBUF):
    fire(b_, b_)

  def loop_body(g, carry):
    s0 = g * NBUF
    for b_ in range(NBUF):
      s = s0 + b_
      compute(s, b_)

      @pl.when(s + NBUF < SPW)
      def _():
        fire(s + NBUF, b_)

    return carry

  lax.fori_loop(0, SPW // NBUF, loop_body, 0)

  pltpu.sync_copy(out_v, out_hbm.at[pl.ds(base, SPW)])


@jax.jit
def kernel(x, vectors):
  vT = vectors.T
  flags = pl.pallas_call(
      _flags_body,
      out_shape=jax.ShapeDtypeStruct((VOCAB,), jnp.float32),
  )(vT)

  u = lax.bitcast_convert_type(vectors, jnp.uint32)
  r = (u + jnp.uint32(0x7FFF) + ((u >> 16) & jnp.uint32(1))) >> 16
  tbi = lax.bitcast_convert_type(r[:, 0::2] | (r[:, 1::2] << 16), jnp.int32)

  mesh = plsc.VectorSubcoreMesh(core_axis_name="c", subcore_axis_name="s")
  run = pl.kernel(
      _body,
      out_type=jax.ShapeDtypeStruct((B, D), jnp.float32),
      mesh=mesh,
      compiler_params=pltpu.CompilerParams(
          needs_layout_passes=False, use_tc_tiling_on_sc=False),
      scratch_types=[
          pltpu.VMEM((L, SPW), jnp.int32),
          pltpu.VMEM((NBUF, LPAD), jnp.int32),
          pltpu.VMEM((NBUF, LGATH, D // 2), jnp.int32),
          pltpu.VMEM((NBUF, LPAD), jnp.float32),
          pltpu.VMEM((SPW, D), jnp.float32),
          pltpu.VMEM_SHARED((VOCAB,), jnp.float32),
      ] + [pltpu.SemaphoreType.DMA] * (2 * NBUF),
  )
  return run(x.T, tbi, flags)


# speedup vs baseline: 5.2275x; 5.2275x over previous
import jax
import jax.numpy as jnp
from jax import lax
from jax.experimental import pallas as pl
from jax.experimental.pallas import tpu as pltpu
from jax.experimental.pallas import tpu_sc as plsc

VOCAB = 100000
B = 16384
L = 50
D = 64
LANES = 16
LPAD = 64
LGATH = 56

NC = 2
NS = 16
NW = NC * NS
SPW = B // NW
NBUF = 8


def _flags_body(v_ref, f_ref):
  s = jnp.sum(v_ref[...], axis=0)
  f_ref[...] = jnp.where(s != 0.0, 1.0, 0.0).astype(jnp.float32)


def _tree_push(stack, v):
  rank = 0
  while stack and stack[-1][0] == rank:
    _, u = stack.pop()
    v = u + v
    rank += 1
  stack.append((rank, v))


def _body(xT_hbm, tab_hbm, flags_hbm, out_hbm,
          idxT_v, sidx_v, rows_v, flg_v, out_v, flags_sh, *sems):
  rsems = sems[:NBUF]
  fsems = sems[NBUF:]
  wid = lax.axis_index("s") * NC + lax.axis_index("c")
  base = wid * SPW

  @pl.when(lax.axis_index("s") == 0)
  def _():
    pltpu.sync_copy(flags_hbm, flags_sh)

  pltpu.sync_copy(xT_hbm.at[:, pl.ds(base, SPW)], idxT_v)

  lane = lax.iota(jnp.int32, LANES)
  zero = jnp.zeros((LANES,), jnp.float32)
  one = jnp.ones((LANES,), jnp.float32)
  zero_i = jnp.zeros((LANES,), jnp.int32)
  rows = [jnp.minimum(k * LANES + lane, L - 1) for k in range(LPAD // LANES)]

  def transpose_body(s, carry):
    col = zero_i + s
    for k in range(LPAD // LANES):
      sidx_v[s, pl.ds(k * LANES, LANES)] = plsc.load_gather(
          idxT_v, [rows[k], col])
    return carry

  lax.fori_loop(0, SPW, transpose_body, 0)
  plsc.subcore_barrier()

  def fire(s, slot):
    sidx = sidx_v.at[s, pl.ds(0, LGATH)]
    pltpu.async_copy(tab_hbm.at[sidx], rows_v.at[slot], rsems[slot])
    pltpu.async_copy(
        flags_sh.at[sidx], flg_v.at[slot, pl.ds(0, LGATH)], fsems[slot])

  def wait(s, slot):
    sidx = sidx_v.at[s, pl.ds(0, LGATH)]
    pltpu.make_async_copy(
        tab_hbm.at[sidx], rows_v.at[slot], rsems[slot]).wait()
    pltpu.make_async_copy(
        flags_sh.at[sidx], flg_v.at[slot, pl.ds(0, LGATH)],
        fsems[slot]).wait()

  def compute(s, slot):
    wait(s, slot)
    flg = flg_v.at[slot]
    g0 = flg[pl.ds(0, LANES)]
    g1 = flg[pl.ds(LANES, LANES)]
    g2 = flg[pl.ds(2 * LANES, LANES)]
    g3 = jnp.where(lane < L - 3 * LANES, flg[pl.ds(3 * LANES, LANES)], zero)
    cs = plsc.cumsum((g0 + g1) + (g2 + g3))
    inv = one / plsc.cummax(lax.rev(cs, (0,)))

    r = rows_v.at[slot]
    stacks = ([], [])
    for l in range(L):
      _tree_push(stacks[0], r[l, pl.ds(0, 32)])
      _tree_push(stacks[1], r[l, pl.ds(32, 32)])
    halves = []
    for st in stacks:
      acc = st[0][1]
      for _, v in st[1:]:
        acc = acc + v
      halves.append(acc)

    orow = out_v.at[s]
    for h in range(2):
      ev, od = plsc.unpack(halves[h], format=plsc.PackFormat.INTERLEAVED)
      plsc.store_scatter(orow, [2 * lane + h * 32], ev * inv)
      plsc.store_scatter(orow, [2 * lane + (h * 32 + 1)], od * inv)

  for b_ in range(NBUF):
    fire(b_, b_)

  def loop_body(g, carry):
    s0 = g * NBUF
    for b_ in range(NBUF):
      s = s0 + b_
      compute(s, b_)

      @pl.when(s + NBUF < SPW)
      def _():
        fire(s + NBUF, b_)

    return carry

  lax.fori_loop(0, SPW // NBUF, loop_body, 0)

  pltpu.sync_copy(out_v, out_hbm.at[pl.ds(base, SPW)])


@jax.jit
def kernel(x, vectors):
  vT = vectors.T
  flags = pl.pallas_call(
      _flags_body,
      out_shape=jax.ShapeDtypeStruct((VOCAB,), jnp.float32),
  )(vT)

  mesh = plsc.VectorSubcoreMesh(core_axis_name="c", subcore_axis_name="s")
  run = pl.kernel(
      _body,
      out_type=jax.ShapeDtypeStruct((B, D), jnp.float32),
      mesh=mesh,
      compiler_params=pltpu.CompilerParams(
          needs_layout_passes=False, use_tc_tiling_on_sc=False),
      scratch_types=[
          pltpu.VMEM((L, SPW), jnp.int32),
          pltpu.VMEM((SPW, LPAD), jnp.int32),
          pltpu.VMEM((NBUF, LGATH, D), jnp.bfloat16),
          pltpu.VMEM((NBUF, LPAD), jnp.float32),
          pltpu.VMEM((SPW, D), jnp.float32),
          pltpu.VMEM_SHARED((VOCAB,), jnp.float32),
      ] + [pltpu.SemaphoreType.DMA] * (2 * NBUF),
  )
  return run(x.T, vectors.astype(jnp.bfloat16), flags)
